# baseline (speedup 1.0000x reference)
import functools

import jax
import jax.numpy as jnp
from jax import lax
from jax.experimental import pallas as pl
from jax.experimental.pallas import tpu as pltpu
from jax.experimental.pallas import tpu_sc as plsc

B = 16384
F = 100
NFEAT = 1000000

NC = 2
NS = 16
NW = NC * NS
ROWS_W = B // NW
CHUNK = 64
NCHUNK = ROWS_W // CHUNK
FP = 128
CW = CHUNK * FP
NJ = CHUNK // 16


def _build_sc_lr():
    mesh = plsc.VectorSubcoreMesh(core_axis_name="c", subcore_axis_name="s")

    @functools.partial(
        pl.kernel,
        mesh=mesh,
        compiler_params=pltpu.CompilerParams(
            needs_layout_passes=False, use_tc_tiling_on_sc=False
        ),
        out_type=jax.ShapeDtypeStruct((B,), jnp.float32),
        scratch_types=[
            pltpu.VMEM((CW,), jnp.int32),
            pltpu.VMEM((CW,), jnp.int32),
            pltpu.VMEM((CW,), jnp.float32),
            pltpu.VMEM((CW,), jnp.float32),
            pltpu.VMEM((CW,), jnp.float32),
            pltpu.VMEM((CW,), jnp.float32),
            pltpu.VMEM((CHUNK,), jnp.float32),
            pltpu.VMEM((8192,), jnp.float32),
            pltpu.VMEM_SHARED((NFEAT,), jnp.float32),
            pltpu.SemaphoreType.DMA,
            pltpu.SemaphoreType.DMA,
        ],
    )
    def k(ids_hbm, vals_hbm, w_hbm, out_hbm,
          ids_v0, ids_v1, vals_v0, vals_v1, emb_v0, emb_v1, acc_v, stg_v,
          w_sh, sem0, sem1):
        wid = lax.axis_index("s") * NC + lax.axis_index("c")
        sid = lax.axis_index("s")
        lane = lax.iota(jnp.int32, 16)
        ids_b = (ids_v0, ids_v1)
        vals_b = (vals_v0, vals_v1)
        emb_b = (emb_v0, emb_v1)
        sem_b = (sem0, sem1)

        PIECE = 8192
        NPIECE = NFEAT // PIECE
        TAIL = NFEAT - NPIECE * PIECE

        def stage(kk, carry):
            p = sid + NS * kk

            @pl.when(p < NPIECE)
            def _copy_piece():
                off = pl.multiple_of(p * PIECE, 128)
                pltpu.sync_copy(w_hbm.at[pl.ds(off, PIECE)], stg_v)
                pltpu.sync_copy(stg_v, w_sh.at[pl.ds(off, PIECE)])

            return carry

        lax.fori_loop(0, (NPIECE + NS - 1) // NS, stage, 0)

        @pl.when(sid == 0)
        def _stage_tail():
            toff = pl.multiple_of(NPIECE * PIECE, 128)
            pltpu.sync_copy(
                w_hbm.at[pl.ds(toff, TAIL)], stg_v.at[pl.ds(0, TAIL)]
            )
            pltpu.sync_copy(
                stg_v.at[pl.ds(0, TAIL)], w_sh.at[pl.ds(toff, TAIL)]
            )

        plsc.subcore_barrier()

        def slab_of(c):
            return pl.multiple_of(wid * (NCHUNK * CW) + c * CW, 128)

        def fire_chunk(c):
            q = c % 2
            slab = slab_of(c)
            pltpu.sync_copy(ids_hbm.at[pl.ds(slab, CW)], ids_b[q])

            pltpu.async_copy(
                w_sh.at[ids_b[q]], emb_b[q], sem_b[q]
            )
            pltpu.sync_copy(vals_hbm.at[pl.ds(slab, CW)], vals_b[q])

        fire_chunk(0)
        for c in range(NCHUNK):
            q = c % 2
            if c + 1 < NCHUNK:
                fire_chunk(c + 1)
            pltpu.make_async_copy(
                out_hbm.at[pl.ds(0, CW)], emb_b[q], sem_b[q]
            ).wait()

            pjs = tuple(lane * FP + j * 16 * FP for j in range(NJ))

            def f_body(f, accs, q=q, pjs=pjs):
                out = []
                for j in range(NJ):
                    p = pjs[j] + f
                    e = plsc.load_gather(emb_b[q], [p])
                    v = plsc.load_gather(vals_b[q], [p])
                    out.append(accs[j] + e * v)
                return tuple(out)

            accs = lax.fori_loop(
                0, F, f_body, (jnp.zeros((16,), jnp.float32),) * NJ
            )
            for j in range(NJ):
                acc_v[pl.ds(j * 16, 16)] = accs[j]
            r0 = pl.multiple_of(wid * ROWS_W + c * CHUNK, CHUNK)
            pltpu.sync_copy(acc_v, out_hbm.at[pl.ds(r0, CHUNK)])

    return k


_SC_LR = _build_sc_lr()


def kernel(ids, vals, weight, bias):
    padblk = (
        jax.lax.broadcasted_iota(jnp.int32, (B, FP - F), 0) * (FP - F)
        + jax.lax.broadcasted_iota(jnp.int32, (B, FP - F), 1)
    )
    ids1 = jnp.concatenate(
        [ids.astype(jnp.int32), padblk], axis=1
    ).reshape(B * FP)
    vals1 = jnp.pad(vals, ((0, 0), (0, FP - F))).reshape(B * FP)
    w1 = weight.reshape(NFEAT)
    y = _SC_LR(ids1, vals1, w1)
    return y + bias


# speedup vs baseline: 1.4240x; 1.4240x over previous
import functools

import jax
import jax.numpy as jnp
from jax import lax
from jax.experimental import pallas as pl
from jax.experimental.pallas import tpu as pltpu
from jax.experimental.pallas import tpu_sc as plsc

B = 16384
F = 100
NFEAT = 1000000

NC = 2
NS = 16
NW = NC * NS
ROWS_W = B // NW
CHUNK = 64
NCHUNK = ROWS_W // CHUNK
CW = CHUNK * F
NROW = CW // 128
NJ = CHUNK // 16


def _build_sc_lr():
    mesh = plsc.VectorSubcoreMesh(core_axis_name="c", subcore_axis_name="s")

    @functools.partial(
        pl.kernel,
        mesh=mesh,
        compiler_params=pltpu.CompilerParams(
            needs_layout_passes=False, use_tc_tiling_on_sc=False
        ),
        out_type=jax.ShapeDtypeStruct((B,), jnp.float32),
        scratch_types=[
            pltpu.VMEM((CW,), jnp.int32),
            pltpu.VMEM((CW,), jnp.int32),
            pltpu.VMEM((CW,), jnp.float32),
            pltpu.VMEM((CW,), jnp.float32),
            pltpu.VMEM((CW,), jnp.float32),
            pltpu.VMEM((CW,), jnp.float32),
            pltpu.VMEM((CHUNK,), jnp.float32),
            pltpu.VMEM((16384,), jnp.float32),
            pltpu.VMEM_SHARED((NFEAT,), jnp.float32),
            pltpu.SemaphoreType.DMA,
            pltpu.SemaphoreType.DMA,
        ],
    )
    def k(ids_hbm, vals_hbm, w_hbm, out_hbm,
          ids_v0, ids_v1, vals_v0, vals_v1, emb_v0, emb_v1, acc_v, stg_v,
          w_sh, sem0, sem1):
        wid = lax.axis_index("s") * NC + lax.axis_index("c")
        sid = lax.axis_index("s")
        lane = lax.iota(jnp.int32, 16)
        ids_b = (ids_v0, ids_v1)
        vals_b = (vals_v0, vals_v1)
        emb_b = (emb_v0, emb_v1)
        sem_b = (sem0, sem1)

        PIECE = 16384
        NPIECE = NFEAT // PIECE
        TAIL = NFEAT - NPIECE * PIECE

        def stage(kk, carry):
            p = sid + NS * kk

            @pl.when(p < NPIECE)
            def _copy_piece():
                off = pl.multiple_of(p * PIECE, 128)
                pltpu.sync_copy(w_hbm.at[pl.ds(off, PIECE)], stg_v)
                pltpu.sync_copy(stg_v, w_sh.at[pl.ds(off, PIECE)])

            return carry

        lax.fori_loop(0, (NPIECE + NS - 1) // NS, stage, 0)

        @pl.when(sid == 0)
        def _stage_tail():
            toff = pl.multiple_of(NPIECE * PIECE, 128)
            pltpu.sync_copy(
                w_hbm.at[pl.ds(toff, TAIL)], stg_v.at[pl.ds(0, TAIL)]
            )
            pltpu.sync_copy(
                stg_v.at[pl.ds(0, TAIL)], w_sh.at[pl.ds(toff, TAIL)]
            )

        plsc.subcore_barrier()

        def slab_of(c):
            return pl.multiple_of(wid * (NCHUNK * CW) + c * CW, 128)

        def fire_chunk(c):
            q = c % 2
            slab = slab_of(c)
            pltpu.sync_copy(ids_hbm.at[pl.ds(slab, CW)], ids_b[q])

            pltpu.async_copy(
                w_sh.at[ids_b[q]], emb_b[q], sem_b[q]
            )
            pltpu.sync_copy(vals_hbm.at[pl.ds(slab, CW)], vals_b[q])

        fire_chunk(0)
        for c in range(NCHUNK):
            q = c % 2
            if c + 1 < NCHUNK:
                fire_chunk(c + 1)
            pltpu.make_async_copy(
                out_hbm.at[pl.ds(0, CW)], emb_b[q], sem_b[q]
            ).wait()

            pjs = tuple(lane * F + j * 16 * F for j in range(NJ))

            def f_body(f, accs, q=q, pjs=pjs):
                out = []
                for j in range(NJ):
                    p = pjs[j] + f
                    e = plsc.load_gather(emb_b[q], [p])
                    v = plsc.load_gather(vals_b[q], [p])
                    out.append(accs[j] + e * v)
                return tuple(out)

            accs = lax.fori_loop(
                0, F, f_body, (jnp.zeros((16,), jnp.float32),) * NJ
            )
            for j in range(NJ):
                acc_v[pl.ds(j * 16, 16)] = accs[j]
            r0 = pl.multiple_of(wid * ROWS_W + c * CHUNK, CHUNK)
            pltpu.sync_copy(acc_v, out_hbm.at[pl.ds(r0, CHUNK)])

    return k


_SC_LR = _build_sc_lr()


def kernel(ids, vals, weight, bias):
    ids1 = ids.astype(jnp.int32).reshape(B * F)
    vals1 = vals.reshape(B * F)
    w1 = weight.reshape(NFEAT)
    y = _SC_LR(ids1, vals1, w1)
    return y + bias


# speedup vs baseline: 1.4245x; 1.0003x over previous
import functools

import jax
import jax.numpy as jnp
from jax import lax
from jax.experimental import pallas as pl
from jax.experimental.pallas import tpu as pltpu
from jax.experimental.pallas import tpu_sc as plsc

B = 16384
F = 100
NFEAT = 1000000

NC = 2
NS = 16
NW = NC * NS
ROWS_W = B // NW
CHUNK = 64
NCHUNK = ROWS_W // CHUNK
CW = CHUNK * F
NJ = CHUNK // 16


def _build_sc_lr():
    mesh = plsc.VectorSubcoreMesh(core_axis_name="c", subcore_axis_name="s")

    @functools.partial(
        pl.kernel,
        mesh=mesh,
        compiler_params=pltpu.CompilerParams(
            needs_layout_passes=False, use_tc_tiling_on_sc=False
        ),
        out_type=jax.ShapeDtypeStruct((B,), jnp.float32),
        scratch_types=[
            pltpu.VMEM((CW,), jnp.int32),
            pltpu.VMEM((CW,), jnp.int32),
            pltpu.VMEM((CW,), jnp.float32),
            pltpu.VMEM((CW,), jnp.float32),
            pltpu.VMEM((CW,), jnp.float32),
            pltpu.VMEM((CW,), jnp.float32),
            pltpu.VMEM((CHUNK,), jnp.float32),
            pltpu.VMEM((16384,), jnp.float32),
            pltpu.VMEM_SHARED((NFEAT,), jnp.float32),
            pltpu.SemaphoreType.DMA,
            pltpu.SemaphoreType.DMA,
        ],
    )
    def k(ids_hbm, vals_hbm, w_hbm, out_hbm,
          ids_v0, ids_v1, vals_v0, vals_v1, emb_v0, emb_v1, acc_v, stg_v,
          w_sh, sem0, sem1):
        wid = lax.axis_index("s") * NC + lax.axis_index("c")
        sid = lax.axis_index("s")
        lane = lax.iota(jnp.int32, 16)
        ids_b = (ids_v0, ids_v1)
        vals_b = (vals_v0, vals_v1)
        emb_b = (emb_v0, emb_v1)
        sem_b = (sem0, sem1)

        PIECE = 16384
        NPIECE = NFEAT // PIECE
        TAIL = NFEAT - NPIECE * PIECE

        def stage(kk, carry):
            p = sid + NS * kk

            @pl.when(p < NPIECE)
            def _copy_piece():
                off = pl.multiple_of(p * PIECE, 128)
                pltpu.sync_copy(w_hbm.at[pl.ds(off, PIECE)], stg_v)
                pltpu.sync_copy(stg_v, w_sh.at[pl.ds(off, PIECE)])

            return carry

        lax.fori_loop(0, (NPIECE + NS - 1) // NS, stage, 0)

        @pl.when(sid == 0)
        def _stage_tail():
            toff = pl.multiple_of(NPIECE * PIECE, 128)
            pltpu.sync_copy(
                w_hbm.at[pl.ds(toff, TAIL)], stg_v.at[pl.ds(0, TAIL)]
            )
            pltpu.sync_copy(
                stg_v.at[pl.ds(0, TAIL)], w_sh.at[pl.ds(toff, TAIL)]
            )

        plsc.subcore_barrier()

        def slab_of(c):
            return pl.multiple_of(wid * (NCHUNK * CW) + c * CW, 128)

        def fire_chunk(c):
            q = c % 2
            slab = slab_of(c)
            pltpu.sync_copy(ids_hbm.at[pl.ds(slab, CW)], ids_b[q])

            pltpu.async_copy(
                w_sh.at[ids_b[q]], emb_b[q], sem_b[q]
            )
            pltpu.sync_copy(vals_hbm.at[pl.ds(slab, CW)], vals_b[q])

        fire_chunk(0)
        for c in range(NCHUNK):
            q = c % 2
            if c + 1 < NCHUNK:
                fire_chunk(c + 1)
            pltpu.make_async_copy(
                out_hbm.at[pl.ds(0, CW)], emb_b[q], sem_b[q]
            ).wait()

            pjs = tuple(lane * F + j * 16 * F for j in range(NJ))

            def f_body(f, accs, q=q, pjs=pjs):
                out = []
                for j in range(NJ):
                    p = pjs[j] + f
                    e = plsc.load_gather(emb_b[q], [p])
                    v = plsc.load_gather(vals_b[q], [p])
                    out.append(accs[j] + e * v)
                return tuple(out)

            accs = lax.fori_loop(
                0, F, f_body, (jnp.zeros((16,), jnp.float32),) * NJ
            )
            for j in range(NJ):
                acc_v[pl.ds(j * 16, 16)] = accs[j]
            r0 = pl.multiple_of(wid * ROWS_W + c * CHUNK, CHUNK)
            pltpu.sync_copy(acc_v, out_hbm.at[pl.ds(r0, CHUNK)])

    return k


_SC_LR = _build_sc_lr()


def kernel(ids, vals, weight, bias):
    ids1 = ids.astype(jnp.int32).reshape(B * F)
    vals1 = vals.reshape(B * F)
    w1 = weight.reshape(NFEAT)
    y = _SC_LR(ids1, vals1, w1)
    return y + bias
